# baseline (speedup 1.0000x reference)
import functools

import jax
import jax.numpy as jnp
from jax import lax
from jax.experimental import pallas as pl
from jax.experimental.pallas import tpu as pltpu
from jax.experimental.pallas import tpu_sc as plsc

BOHR = 0.52917721092
INV_BOHR = 1.0 / BOHR
N_NODES = 50000
N_EDGES = 1600000

NC = 2
NS = 16
NW = NC * NS
CHUNK_ROWS = 8
ROW = 128
CHUNK = CHUNK_ROWS * ROW
N_CHUNKS = 50
PER_W = CHUNK * N_CHUNKS
E_PAD = PER_W * NW
ROWS_TOTAL = E_PAD // ROW
ROWS_ALLOC = ROWS_TOTAL + CHUNK_ROWS
ACC_PAD = 50176
SLICE = ACC_PAD // NS


def _zbl_sc_kernel(ztab_hbm, zptab_hbm, coef_hbm, zeros_hbm,
                   src_hbm, dst_hbm, dist_hbm, sw_hbm,
                   out_hbm,
                   ztab, zptab, coef,
                   srcb, dstb, distb, swb, vals, sbuf,
                   acc,
                   sem_in, sem_sc):
    cid = lax.axis_index("c")
    sid = lax.axis_index("s")
    wid = sid * NC + cid

    pltpu.sync_copy(ztab_hbm, ztab)
    pltpu.sync_copy(zptab_hbm, zptab)
    pltpu.sync_copy(coef_hbm, coef)

    pltpu.sync_copy(zeros_hbm, sbuf)
    pltpu.sync_copy(sbuf, acc.at[pl.ds(sid * SLICE, SLICE)])
    plsc.subcore_barrier()

    c0 = coef[0, :]
    c1 = coef[1, :]
    c2 = coef[2, :]
    c3 = coef[3, :]
    na0 = coef[4, :]
    na1 = coef[5, :]
    na2 = coef[6, :]
    na3 = coef[7, :]

    row_base = wid * (N_CHUNKS * CHUNK_ROWS)

    def fire_inputs(ch, slot):
        r0 = row_base + ch * CHUNK_ROWS
        rs = pl.ds(r0, CHUNK_ROWS)
        pltpu.async_copy(src_hbm.at[rs], srcb.at[slot], sem_in.at[slot])
        pltpu.async_copy(dst_hbm.at[rs], dstb.at[slot], sem_in.at[slot])
        pltpu.async_copy(dist_hbm.at[rs], distb.at[slot], sem_in.at[slot])
        pltpu.async_copy(sw_hbm.at[rs], swb.at[slot], sem_in.at[slot])

    def wait_inputs(ch, slot):
        r0 = row_base + ch * CHUNK_ROWS
        rs = pl.ds(r0, CHUNK_ROWS)
        pltpu.make_async_copy(src_hbm.at[rs], srcb.at[slot], sem_in.at[slot]).wait()
        pltpu.make_async_copy(dst_hbm.at[rs], dstb.at[slot], sem_in.at[slot]).wait()
        pltpu.make_async_copy(dist_hbm.at[rs], distb.at[slot], sem_in.at[slot]).wait()
        pltpu.make_async_copy(sw_hbm.at[rs], swb.at[slot], sem_in.at[slot]).wait()

    def compute_chunk(slot):
        prev = None
        for j in range(CHUNK_ROWS):
            for k in range(ROW // 16):
                sl = pl.ds(k * 16, 16)
                si = srcb[slot, j, sl]
                di = dstb[slot, j, sl]
                zi = plsc.load_gather(ztab, [si])
                zj = plsc.load_gather(ztab, [di])
                zpi = plsc.load_gather(zptab, [si])
                zpj = plsc.load_gather(zptab, [di])
                dist = distb[slot, j, sl]
                x = dist * (zpi + zpj)
                phi = (jnp.exp(na0 * x + c0) + jnp.exp(na1 * x + c1)
                       + jnp.exp(na2 * x + c2) + jnp.exp(na3 * x + c3))
                vals[slot, j, sl] = zi * zj * phi * swb[slot, j, sl] / dist
            if prev is not None:
                prev.wait()
            prev = pltpu.async_copy(vals.at[slot, j], acc.at[srcb.at[slot, j]],
                                    sem_sc.at[slot], add=True)
        prev.wait()

    fire_inputs(0, 0)

    def pair_body(i, carry):
        ch0 = i * 2
        fire_inputs(ch0 + 1, 1)
        wait_inputs(ch0, 0)
        compute_chunk(0)
        fire_inputs(ch0 + 2, 0)
        wait_inputs(ch0 + 1, 1)
        compute_chunk(1)
        return carry

    lax.fori_loop(0, N_CHUNKS // 2, pair_body, None)
    wait_inputs(N_CHUNKS, 0)

    plsc.subcore_barrier()
    pltpu.sync_copy(acc.at[pl.ds(sid * SLICE, SLICE)], sbuf)
    pltpu.sync_copy(sbuf,
                    out_hbm.at[pl.ds(cid * ACC_PAD + sid * SLICE, SLICE)])


_zbl_sc = functools.partial(
    pl.kernel,
    out_type=jax.ShapeDtypeStruct((NC * ACC_PAD,), jnp.float32),
    mesh=plsc.VectorSubcoreMesh(core_axis_name="c", subcore_axis_name="s"),
    compiler_params=pltpu.CompilerParams(needs_layout_passes=False),
    scratch_types=[
        pltpu.VMEM((N_NODES,), jnp.float32),
        pltpu.VMEM((N_NODES,), jnp.float32),
        pltpu.VMEM((8, 16), jnp.float32),
        pltpu.VMEM((2, CHUNK_ROWS, ROW), jnp.int32),
        pltpu.VMEM((2, CHUNK_ROWS, ROW), jnp.int32),
        pltpu.VMEM((2, CHUNK_ROWS, ROW), jnp.float32),
        pltpu.VMEM((2, CHUNK_ROWS, ROW), jnp.float32),
        pltpu.VMEM((2, CHUNK_ROWS, ROW), jnp.float32),
        pltpu.VMEM((SLICE,), jnp.float32),
        pltpu.VMEM_SHARED((ACC_PAD,), jnp.float32),
        pltpu.SemaphoreType.DMA((2,)),
        pltpu.SemaphoreType.DMA((2,)),
    ],
)(_zbl_sc_kernel)


def _combine_body(p_ref, o_ref):
    o_ref[...] = p_ref[0] + p_ref[1]


def _combine(partials):
    return pl.pallas_call(
        _combine_body,
        out_shape=jax.ShapeDtypeStruct((ACC_PAD // 128, 128), jnp.float32),
    )(partials.reshape(NC, ACC_PAD // 128, 128))


def kernel(species, edge_src, edge_dst, distances, switch,
           d_param, p_param, cs_param, alphas_param):
    f32 = jnp.float32
    d = jnp.abs(d_param).astype(f32)
    p = jnp.abs(p_param).astype(f32)
    cs = 0.5 * jax.nn.softmax(cs_param.astype(f32))
    alphas = jnp.abs(alphas_param).astype(f32)

    Z = jnp.where(species > 0, species.astype(f32), 0.0)
    Zp = Z ** p / d

    cs_f = jnp.log(cs * BOHR)
    al_f = -alphas * INV_BOHR
    coef = jnp.broadcast_to(
        jnp.concatenate([cs_f, al_f])[:, None], (8, 16)).astype(f32)
    zeros = jnp.zeros((SLICE,), f32)

    pad = ROWS_ALLOC * ROW - N_EDGES
    src = jnp.concatenate(
        [edge_src.astype(jnp.int32), jnp.zeros((pad,), jnp.int32)]
    ).reshape(-1, ROW)
    dst = jnp.concatenate(
        [edge_dst.astype(jnp.int32), jnp.zeros((pad,), jnp.int32)]
    ).reshape(-1, ROW)
    dist = jnp.concatenate(
        [distances.astype(f32), jnp.ones((pad,), f32)]
    ).reshape(-1, ROW)
    sw = jnp.concatenate(
        [switch.astype(f32), jnp.zeros((pad,), f32)]
    ).reshape(-1, ROW)

    partials = _zbl_sc(Z, Zp, coef, zeros, src, dst, dist, sw)
    summed = _combine(partials)
    return summed.reshape(-1)[:N_NODES]


# speedup vs baseline: 1.9385x; 1.9385x over previous
import functools

import jax
import jax.numpy as jnp
from jax import lax
from jax.experimental import pallas as pl
from jax.experimental.pallas import tpu as pltpu
from jax.experimental.pallas import tpu_sc as plsc

BOHR = 0.52917721092
INV_BOHR = 1.0 / BOHR
N_NODES = 50000
N_EDGES = 1600000

NC = 2
NS = 16
NW = NC * NS
CHUNK_ROWS = 8
ROW = 128
CHUNK = CHUNK_ROWS * ROW
N_CHUNKS = 50
PER_W = CHUNK * N_CHUNKS
E_PAD = PER_W * NW
ROWS_TOTAL = E_PAD // ROW
ROWS_ALLOC = ROWS_TOTAL + 2 * CHUNK_ROWS
ACC_PAD = 50176
SLICE = ACC_PAD // NS


def _zbl_sc_kernel(ztab_hbm, zptab_hbm, coef_hbm, zeros_hbm,
                   src_hbm, dst_hbm, dist_hbm, sw_hbm, srcr_hbm,
                   out_hbm,
                   ztab, zptab, coef,
                   srcb, dstb, distb, swb, vals, sidx, sbuf,
                   acc,
                   sem_in, sem_si, sem_sc):
    cid = lax.axis_index("c")
    sid = lax.axis_index("s")
    wid = sid * NC + cid

    pltpu.sync_copy(ztab_hbm, ztab)
    pltpu.sync_copy(zptab_hbm, zptab)
    pltpu.sync_copy(coef_hbm, coef)

    pltpu.sync_copy(zeros_hbm, sbuf)
    pltpu.sync_copy(sbuf, acc.at[pl.ds(sid * SLICE, SLICE)])
    plsc.subcore_barrier()

    c0 = coef[0, :]
    c1 = coef[1, :]
    c2 = coef[2, :]
    c3 = coef[3, :]
    na0 = coef[4, :]
    na1 = coef[5, :]
    na2 = coef[6, :]
    na3 = coef[7, :]

    row_base = wid * (N_CHUNKS * CHUNK_ROWS)
    elem_base = row_base * ROW

    def fire_inputs(ch, slot):
        es = pl.ds(elem_base + ch * CHUNK, CHUNK)
        pltpu.async_copy(src_hbm.at[es], srcb.at[slot], sem_in.at[slot])
        pltpu.async_copy(dst_hbm.at[es], dstb.at[slot], sem_in.at[slot])
        pltpu.async_copy(dist_hbm.at[es], distb.at[slot], sem_in.at[slot])
        pltpu.async_copy(sw_hbm.at[es], swb.at[slot], sem_in.at[slot])

    def wait_inputs(ch, slot):
        es = pl.ds(elem_base + ch * CHUNK, CHUNK)
        pltpu.make_async_copy(src_hbm.at[es], srcb.at[slot], sem_in.at[slot]).wait()
        pltpu.make_async_copy(dst_hbm.at[es], dstb.at[slot], sem_in.at[slot]).wait()
        pltpu.make_async_copy(dist_hbm.at[es], distb.at[slot], sem_in.at[slot]).wait()
        pltpu.make_async_copy(sw_hbm.at[es], swb.at[slot], sem_in.at[slot]).wait()

    def drain_scatters(slot):
        for j in range(CHUNK_ROWS):
            pltpu.make_async_copy(
                vals.at[slot, pl.ds(j * ROW, ROW)],
                acc.at[sidx.at[slot, j]],
                sem_sc.at[slot, j]).wait()

    def turn(i, ch, slot):
        wait_inputs(ch, slot)

        @pl.when(i > 0)
        def _():
            drain_scatters(slot)

        rs = pl.ds(row_base + ch * CHUNK_ROWS, CHUNK_ROWS)
        sidx_cp = pltpu.async_copy(srcr_hbm.at[rs], sidx.at[slot],
                                   sem_si.at[slot])

        @plsc.parallel_loop(0, CHUNK, 16, unroll=4)
        def _(off):
            sl = pl.ds(off, 16)
            si = srcb[slot, sl]
            di = dstb[slot, sl]
            zi = plsc.load_gather(ztab, [si])
            zj = plsc.load_gather(ztab, [di])
            zpi = plsc.load_gather(zptab, [si])
            zpj = plsc.load_gather(zptab, [di])
            dist = distb[slot, sl]
            x = dist * (zpi + zpj)
            phi = (jnp.exp(na0 * x + c0) + jnp.exp(na1 * x + c1)
                   + jnp.exp(na2 * x + c2) + jnp.exp(na3 * x + c3))
            vals[slot, sl] = zi * zj * phi * swb[slot, sl] / dist

        sidx_cp.wait()
        for j in range(CHUNK_ROWS):
            pltpu.async_copy(vals.at[slot, pl.ds(j * ROW, ROW)],
                             acc.at[sidx.at[slot, j]],
                             sem_sc.at[slot, j], add=True)
        fire_inputs(ch + 2, slot)

    fire_inputs(0, 0)
    fire_inputs(1, 1)

    def pair_body(i, carry):
        ch0 = i * 2
        turn(i, ch0, 0)
        turn(i, ch0 + 1, 1)
        return carry

    lax.fori_loop(0, N_CHUNKS // 2, pair_body, None)

    drain_scatters(0)
    drain_scatters(1)
    wait_inputs(N_CHUNKS, 0)
    wait_inputs(N_CHUNKS + 1, 1)

    plsc.subcore_barrier()
    pltpu.sync_copy(acc.at[pl.ds(sid * SLICE, SLICE)], sbuf)
    pltpu.sync_copy(sbuf,
                    out_hbm.at[pl.ds(cid * ACC_PAD + sid * SLICE, SLICE)])


_zbl_sc = functools.partial(
    pl.kernel,
    out_type=jax.ShapeDtypeStruct((NC * ACC_PAD,), jnp.float32),
    mesh=plsc.VectorSubcoreMesh(core_axis_name="c", subcore_axis_name="s"),
    compiler_params=pltpu.CompilerParams(needs_layout_passes=False),
    scratch_types=[
        pltpu.VMEM((N_NODES,), jnp.float32),
        pltpu.VMEM((N_NODES,), jnp.float32),
        pltpu.VMEM((8, 16), jnp.float32),
        pltpu.VMEM((2, CHUNK), jnp.int32),
        pltpu.VMEM((2, CHUNK), jnp.int32),
        pltpu.VMEM((2, CHUNK), jnp.float32),
        pltpu.VMEM((2, CHUNK), jnp.float32),
        pltpu.VMEM((2, CHUNK), jnp.float32),
        pltpu.VMEM((2, CHUNK_ROWS, ROW), jnp.int32),
        pltpu.VMEM((SLICE,), jnp.float32),
        pltpu.VMEM_SHARED((ACC_PAD,), jnp.float32),
        pltpu.SemaphoreType.DMA((2,)),
        pltpu.SemaphoreType.DMA((2,)),
        pltpu.SemaphoreType.DMA((2, CHUNK_ROWS)),
    ],
)(_zbl_sc_kernel)


def _combine_body(p_ref, o_ref):
    o_ref[...] = p_ref[0] + p_ref[1]


def _combine(partials):
    return pl.pallas_call(
        _combine_body,
        out_shape=jax.ShapeDtypeStruct((ACC_PAD // 128, 128), jnp.float32),
    )(partials.reshape(NC, ACC_PAD // 128, 128))


def kernel(species, edge_src, edge_dst, distances, switch,
           d_param, p_param, cs_param, alphas_param):
    f32 = jnp.float32
    d = jnp.abs(d_param).astype(f32)
    p = jnp.abs(p_param).astype(f32)
    cs = 0.5 * jax.nn.softmax(cs_param.astype(f32))
    alphas = jnp.abs(alphas_param).astype(f32)

    Z = jnp.where(species > 0, species.astype(f32), 0.0)
    Zp = Z ** p / d

    cs_f = jnp.log(cs * BOHR)
    al_f = -alphas * INV_BOHR
    coef = jnp.broadcast_to(
        jnp.concatenate([cs_f, al_f])[:, None], (8, 16)).astype(f32)
    zeros = jnp.zeros((SLICE,), f32)

    pad = ROWS_ALLOC * ROW - N_EDGES
    src = jnp.concatenate(
        [edge_src.astype(jnp.int32), jnp.zeros((pad,), jnp.int32)])
    dst = jnp.concatenate(
        [edge_dst.astype(jnp.int32), jnp.zeros((pad,), jnp.int32)])
    dist = jnp.concatenate(
        [distances.astype(f32), jnp.ones((pad,), f32)])
    sw = jnp.concatenate(
        [switch.astype(f32), jnp.zeros((pad,), f32)])
    src_rows = src.reshape(-1, ROW)

    partials = _zbl_sc(Z, Zp, coef, zeros, src, dst, dist, sw, src_rows)
    summed = _combine(partials)
    return summed.reshape(-1)[:N_NODES]
